# baseline (speedup 1.0000x reference)
import functools

import jax
import jax.numpy as jnp
from jax import lax
from jax.experimental import pallas as pl
from jax.experimental.pallas import tpu as pltpu
from jax.experimental.pallas import tpu_sc as plsc

N = 10000
E = 320000
D = 128
H = 128

NC = 2
NS = 16
NW = NC * NS
L = 16

NP = 10240
EPT = E // NW
C = 80
NCHUNK = EPT // C
CA = 400
NCA = EPT // CA

TCB = 2000


def _tc1_body(x_ref, w_ref, wp_ref, a_ref, b_ref, sfts_ref, sfb_ref,
              f12_ref, m_ref):
    i = pl.program_id(0)
    s = jnp.dot(x_ref[...], w_ref[...],
                precision=lax.Precision.HIGHEST,
                preferred_element_type=jnp.float32)
    sfts_ref[...] = s
    sp = jnp.dot(x_ref[...], wp_ref[...],
                 precision=lax.Precision.HIGHEST,
                 preferred_element_type=jnp.float32)
    sfb_ref[...] = sp.astype(jnp.bfloat16)
    f = jnp.dot(s, a_ref[...],
                precision=lax.Precision.HIGHEST,
                preferred_element_type=jnp.float32) + b_ref[...]
    f12_ref[...] = f
    m = jnp.max(f, axis=0, keepdims=True)

    @pl.when(i == 0)
    def _():
        m_ref[...] = m

    @pl.when(i != 0)
    def _():
        m_ref[...] = jnp.maximum(m_ref[...], m)


def _tc1(x, W, Wp, A, b2d):
    return pl.pallas_call(
        _tc1_body,
        grid=(N // TCB,),
        in_specs=[
            pl.BlockSpec((TCB, D), lambda i: (i, 0)),
            pl.BlockSpec((D, H), lambda i: (0, 0)),
            pl.BlockSpec((D, H), lambda i: (0, 0)),
            pl.BlockSpec((H, 2), lambda i: (0, 0)),
            pl.BlockSpec((1, 2), lambda i: (0, 0)),
        ],
        out_specs=[
            pl.BlockSpec((TCB, H), lambda i: (i, 0)),
            pl.BlockSpec((TCB, H), lambda i: (i, 0)),
            pl.BlockSpec((TCB, 2), lambda i: (i, 0)),
            pl.BlockSpec((1, 2), lambda i: (0, 0)),
        ],
        out_shape=[
            jax.ShapeDtypeStruct((N, H), jnp.float32),
            jax.ShapeDtypeStruct((N, H), jnp.bfloat16),
            jax.ShapeDtypeStruct((N, 2), jnp.float32),
            jax.ShapeDtypeStruct((1, 2), jnp.float32),
        ],
    )(x, W, Wp, A, b2d)


_MESH = plsc.VectorSubcoreMesh(
    core_axis_name="c", subcore_axis_name="s", num_cores=NC, num_subcores=NS)

_SC_PARAMS = pltpu.CompilerParams(
    needs_layout_passes=False, use_tc_tiling_on_sc=False)


def _sc1b_body(rows_hbm, cols_hbm, f1_hbm, f2_hbm, m_hbm, sfb_hbm,
               vals0_out, vals1_out, den0_out, den1_out,
               rv0, rv1, cv0, cv1, fa0, fa1, fb0, fb1, m_v,
               rs0, rs1, bh0, bh1, gf0, gf1, x0, x1,
               vals_sh, den16_sh, gsem, isem, fsem, ssem0, ssem1):
    core = lax.axis_index("c")
    sid = lax.axis_index("s")
    tbase = (core * NS + sid) * EPT
    RV = [rv0, rv1]
    CV = [cv0, cv1]
    FA = [fa0, fa1]
    FB = [fb0, fb1]
    RS = [rs0, rs1]
    BH = [bh0, bh1]
    GF = [gf0, gf1]
    X16 = [x0, x1]
    SS = [ssem0, ssem1]
    zeros16f = jnp.zeros((L,), jnp.float32)
    zeros16i = jnp.zeros((L,), jnp.int32)
    iota16 = lax.iota(jnp.int32, L)

    def _z1(r, _):
        for j in range(D // L):
            gf0[r, pl.ds(j * L, L)] = zeros16f
        x0[r, :] = zeros16f
        return 0
    lax.fori_loop(0, C, _z1, 0)
    drows = NP // NS
    for k in range(drows // C):
        pltpu.sync_copy(gf0, vals_sh.at[pl.ds(sid * drows + k * C, C), :])
        pltpu.sync_copy(x0, den16_sh.at[pl.ds(sid * drows + k * C, C), :])
    plsc.subcore_barrier()

    def issue_idx(k, P):
        sl = pl.ds(tbase + k * C, C)
        pltpu.async_copy(rows_hbm.at[sl], RV[P], isem)
        pltpu.async_copy(cols_hbm.at[sl], CV[P], isem)

    def wait_idx(k, P):
        sl = pl.ds(tbase + k * C, C)
        pltpu.make_async_copy(rows_hbm.at[sl], RV[P], isem).wait()
        pltpu.make_async_copy(cols_hbm.at[sl], CV[P], isem).wait()

    def issue_fg(P):
        pltpu.async_copy(f1_hbm.at[RV[P]], FA[P], fsem)
        pltpu.async_copy(f2_hbm.at[CV[P]], FB[P], fsem)

    def wait_fg(P):
        pltpu.make_async_copy(f1_hbm.at[RV[P]], FA[P], fsem).wait()
        pltpu.make_async_copy(f2_hbm.at[CV[P]], FB[P], fsem).wait()

    def wait_scatter(P):
        pltpu.make_async_copy(GF[P], vals_sh.at[RS[P]], SS[P]).wait()
        pltpu.make_async_copy(X16[P], den16_sh.at[RS[P]], SS[P]).wait()

    pltpu.sync_copy(m_hbm, m_v)
    mrow = m_v[:]
    c_shift = jnp.maximum(mrow[0] + mrow[1], 0.0)

    issue_idx(0, 0)
    issue_idx(1, 1)
    wait_idx(0, 0)
    issue_fg(0)
    pltpu.async_copy(sfb_hbm.at[CV[0]], BH[0], gsem)

    def _body(k, P, Q):
        pltpu.make_async_copy(sfb_hbm.at[CV[P]], BH[P], gsem).wait()
        wait_fg(P)

        @pl.when(k >= 2)
        def _():
            wait_scatter(P)

        for g in range(C // L):
            sl = pl.ds(g * L, L)
            RS[P][sl] = RV[P][sl]

        @pl.when(k + 1 < NCHUNK)
        def _():
            wait_idx(k + 1, Q)
            issue_fg(Q)

        @pl.when(k + 2 < NCHUNK)
        def _():
            issue_idx(k + 2, P)

        @pl.when(k + 1 < NCHUNK)
        def _():
            pltpu.async_copy(sfb_hbm.at[CV[Q]], BH[Q], gsem)

        def _grp(g, _):
            sl16 = pl.ds(g * L, L)
            e = FA[P][sl16] + FB[P][sl16]
            e = jnp.where(e >= 0.0, e, 0.2 * e) - c_shift
            exvec = jnp.exp(e)
            plsc.store_scatter(X16[P], [iota16 + g * L, zeros16i], exvec)
            for lane in range(L):
                wv = jnp.full((L,), exvec[lane], jnp.float32)
                ei = g * L + lane
                for j in range(D // (2 * L)):
                    v32 = BH[P][ei, pl.ds(j * 2 * L, 2 * L)]
                    a, b = plsc.unpack(v32, format=plsc.PackFormat.INTERLEAVED)
                    GF[P][ei, pl.ds(j * 2 * L, L)] = a * wv
                    GF[P][ei, pl.ds(j * 2 * L + L, L)] = b * wv
            return 0
        lax.fori_loop(0, C // L, _grp, 0)

        pltpu.async_copy(GF[P], vals_sh.at[RS[P]], SS[P], add=True)
        pltpu.async_copy(X16[P], den16_sh.at[RS[P]], SS[P], add=True)

    def _full(k, _):
        for ph in range(2):
            @pl.when(k % 2 == ph)
            def _():
                _body(k, ph, 1 - ph)
        return 0
    lax.fori_loop(0, NCHUNK, _full, 0)

    wait_scatter((NCHUNK - 2) % 2)
    wait_scatter((NCHUNK - 1) % 2)
    plsc.subcore_barrier()

    for kk in range(drows // C):
        pltpu.sync_copy(den16_sh.at[pl.ds(sid * drows + kk * C, C), :], x0)
        for g in range(C // L):
            dv = plsc.load_gather(x0, [iota16 + g * L, zeros16i])
            for lane in range(L):
                ei = g * L + lane
                wv = jnp.full((L,), dv[lane], jnp.float32)
                for j in range(D // L):
                    gf0[ei, pl.ds(j * L, L)] = wv

        @pl.when(core == 0)
        def _():
            pltpu.sync_copy(
                gf0, den0_out.at[pl.ds(sid * drows + kk * C, C), :])

        @pl.when(core == 1)
        def _():
            pltpu.sync_copy(
                gf0, den1_out.at[pl.ds(sid * drows + kk * C, C), :])

    @pl.when(core == 0)
    def _():
        pltpu.sync_copy(vals_sh.at[pl.ds(sid * drows, drows), :],
                        vals0_out.at[pl.ds(sid * drows, drows), :])

    @pl.when(core == 1)
    def _():
        pltpu.sync_copy(vals_sh.at[pl.ds(sid * drows, drows), :],
                        vals1_out.at[pl.ds(sid * drows, drows), :])


_sc1b = functools.partial(
    pl.kernel, _sc1b_body,
    compiler_params=_SC_PARAMS,
    out_type=(
        jax.ShapeDtypeStruct((NP, H), jnp.float32),
        jax.ShapeDtypeStruct((NP, H), jnp.float32),
        jax.ShapeDtypeStruct((NP, H), jnp.float32),
        jax.ShapeDtypeStruct((NP, H), jnp.float32),
    ),
    mesh=_MESH,
    scratch_types=(
        pltpu.VMEM((C,), jnp.int32),
        pltpu.VMEM((C,), jnp.int32),
        pltpu.VMEM((C,), jnp.int32),
        pltpu.VMEM((C,), jnp.int32),
        pltpu.VMEM((C,), jnp.float32),
        pltpu.VMEM((C,), jnp.float32),
        pltpu.VMEM((C,), jnp.float32),
        pltpu.VMEM((C,), jnp.float32),
        pltpu.VMEM((L,), jnp.float32),
        pltpu.VMEM((C,), jnp.int32),
        pltpu.VMEM((C,), jnp.int32),
        pltpu.VMEM((C, H), jnp.bfloat16),
        pltpu.VMEM((C, H), jnp.bfloat16),
        pltpu.VMEM((C, D), jnp.float32),
        pltpu.VMEM((C, D), jnp.float32),
        pltpu.VMEM((C, L), jnp.float32),
        pltpu.VMEM((C, L), jnp.float32),
        pltpu.VMEM_SHARED((NP, H), jnp.float32),
        pltpu.VMEM_SHARED((NP, L), jnp.float32),
        pltpu.SemaphoreType.DMA,
        pltpu.SemaphoreType.DMA,
        pltpu.SemaphoreType.DMA,
        pltpu.SemaphoreType.DMA,
        pltpu.SemaphoreType.DMA,
    ),
)()


def _tc2_body(v0_ref, v1_ref, de0_ref, de1_ref, bias_ref, out_ref):
    v = v0_ref[...] + v1_ref[...]
    d = de0_ref[...] + de1_ref[...]
    o = v * jnp.where(d != 0.0, 1.0 / d, 0.0) + bias_ref[...]
    out_ref[0] = jnp.where(o > 0.0, o, jnp.exp(jnp.minimum(o, 0.0)) - 1.0)


def _tc2(vals0, vals1, den0, den1, bias2d):
    return pl.pallas_call(
        _tc2_body,
        grid=(N // TCB,),
        in_specs=[
            pl.BlockSpec((TCB, H), lambda i: (i, 0)),
            pl.BlockSpec((TCB, H), lambda i: (i, 0)),
            pl.BlockSpec((TCB, H), lambda i: (i, 0)),
            pl.BlockSpec((TCB, H), lambda i: (i, 0)),
            pl.BlockSpec((1, H), lambda i: (0, 0)),
        ],
        out_specs=pl.BlockSpec((1, TCB, H), lambda i: (0, i, 0)),
        out_shape=jax.ShapeDtypeStruct((1, N, H), jnp.float32),
    )(vals0, vals1, den0, den1, bias2d)


def kernel(seq, edge_index, training, W, a1, b1, a2, b2, bias_zero):
    x = seq[0]
    rows = edge_index[0]
    cols = edge_index[1]
    A = jnp.concatenate([a1, a2], axis=1)
    b2d = jnp.concatenate([b1, b2]).reshape(1, 2)
    perm = []
    for j in range(H // 32):
        for i in range(16):
            perm += [j * 32 + i, j * 32 + 16 + i]
    Wp = W[:, jnp.array(perm, dtype=jnp.int32)]
    sfts, sfb, f12, m12 = _tc1(x, W, Wp, A, b2d)
    f1 = f12[:, 0]
    f2 = f12[:, 1]
    mpad = jnp.pad(m12.reshape(2), (0, L - 2))
    vals0, vals1, den0, den1 = _sc1b(rows, cols, f1, f2, mpad, sfb)
    return _tc2(vals0[:N], vals1[:N], den0[:N], den1[:N],
                bias_zero.reshape(1, H))


# speedup vs baseline: 1.3869x; 1.3869x over previous
import functools

import jax
import jax.numpy as jnp
from jax import lax
from jax.experimental import pallas as pl
from jax.experimental.pallas import tpu as pltpu
from jax.experimental.pallas import tpu_sc as plsc

N = 10000
E = 320000
D = 128
H = 128

NC = 2
NS = 16
NW = NC * NS
L = 16

NP = 10240
EPT = E // NW
C = 80
NCHUNK = EPT // C
CA = 400
NCA = EPT // CA

TCB = 2000


def _tc1_body(x_ref, w_ref, a_ref, b_ref, sfts_ref, f12_ref, m_ref):
    i = pl.program_id(0)
    s = jnp.dot(x_ref[...], w_ref[...],
                precision=lax.Precision.HIGHEST,
                preferred_element_type=jnp.float32)
    sfts_ref[...] = s
    f = jnp.dot(s, a_ref[...],
                precision=lax.Precision.HIGHEST,
                preferred_element_type=jnp.float32) + b_ref[...]
    f12_ref[...] = f
    m = jnp.max(f, axis=0, keepdims=True)

    @pl.when(i == 0)
    def _():
        m_ref[...] = m

    @pl.when(i != 0)
    def _():
        m_ref[...] = jnp.maximum(m_ref[...], m)


def _tc1(x, W, A, b2d):
    return pl.pallas_call(
        _tc1_body,
        grid=(N // TCB,),
        in_specs=[
            pl.BlockSpec((TCB, D), lambda i: (i, 0)),
            pl.BlockSpec((D, H), lambda i: (0, 0)),
            pl.BlockSpec((H, 2), lambda i: (0, 0)),
            pl.BlockSpec((1, 2), lambda i: (0, 0)),
        ],
        out_specs=[
            pl.BlockSpec((TCB, H), lambda i: (i, 0)),
            pl.BlockSpec((TCB, 2), lambda i: (i, 0)),
            pl.BlockSpec((1, 2), lambda i: (0, 0)),
        ],
        out_shape=[
            jax.ShapeDtypeStruct((N, H), jnp.float32),
            jax.ShapeDtypeStruct((N, 2), jnp.float32),
            jax.ShapeDtypeStruct((1, 2), jnp.float32),
        ],
    )(x, W, A, b2d)


_MESH = plsc.VectorSubcoreMesh(
    core_axis_name="c", subcore_axis_name="s", num_cores=NC, num_subcores=NS)

_SC_PARAMS = pltpu.CompilerParams(
    needs_layout_passes=False, use_tc_tiling_on_sc=False)


def _sc1b_body(rows_hbm, cols_hbm, f1_hbm, f2_hbm, m_hbm, sfts_hbm,
               vals0_out, vals1_out, den0_out, den1_out,
               rv0, rv1, rv2, cv0, cv1, cv2,
               fa0, fa1, fa2, fb0, fb1, fb2, m_v,
               rs0, rs1, rs2, g0, g1, g2, x0, x1, x2, dent,
               vals_sh, den16_sh, gsem, isem, fsem, ssem0, ssem1, ssem2):
    core = lax.axis_index("c")
    sid = lax.axis_index("s")
    tbase = (core * NS + sid) * EPT
    RV = [rv0, rv1, rv2]
    CV = [cv0, cv1, cv2]
    FA = [fa0, fa1, fa2]
    FB = [fb0, fb1, fb2]
    RS = [rs0, rs1, rs2]
    G = [g0, g1, g2]
    X16 = [x0, x1, x2]
    SS = [ssem0, ssem1, ssem2]
    zeros16f = jnp.zeros((L,), jnp.float32)
    zeros16i = jnp.zeros((L,), jnp.int32)
    iota16 = lax.iota(jnp.int32, L)

    def _z1(r, _):
        for j in range(D // L):
            g0[r, pl.ds(j * L, L)] = zeros16f
        x0[r, :] = zeros16f
        return 0
    lax.fori_loop(0, C, _z1, 0)
    rows_per_tile = NP // NS
    for k in range(rows_per_tile // C):
        pltpu.sync_copy(g0, vals_sh.at[pl.ds(sid * rows_per_tile + k * C, C), :])
        pltpu.sync_copy(x0, den16_sh.at[pl.ds(sid * rows_per_tile + k * C, C), :])
    plsc.subcore_barrier()

    def issue_idx(k, P):
        sl = pl.ds(tbase + k * C, C)
        pltpu.async_copy(rows_hbm.at[sl], RV[P], isem)
        pltpu.async_copy(cols_hbm.at[sl], CV[P], isem)

    def wait_idx(k, P):
        sl = pl.ds(tbase + k * C, C)
        pltpu.make_async_copy(rows_hbm.at[sl], RV[P], isem).wait()
        pltpu.make_async_copy(cols_hbm.at[sl], CV[P], isem).wait()

    def issue_fg(P):
        pltpu.async_copy(f1_hbm.at[RV[P]], FA[P], fsem)
        pltpu.async_copy(f2_hbm.at[CV[P]], FB[P], fsem)

    def wait_fg(P):
        pltpu.make_async_copy(f1_hbm.at[RV[P]], FA[P], fsem).wait()
        pltpu.make_async_copy(f2_hbm.at[CV[P]], FB[P], fsem).wait()

    def wait_scatter(R):
        pltpu.make_async_copy(G[R], vals_sh.at[RS[R]], SS[R]).wait()
        pltpu.make_async_copy(X16[R], den16_sh.at[RS[R]], SS[R]).wait()

    pltpu.sync_copy(m_hbm, m_v)
    mrow = m_v[:]
    c_shift = jnp.maximum(mrow[0] + mrow[1], 0.0)

    issue_idx(0, 0)
    issue_idx(1, 1)
    wait_idx(0, 0)
    issue_fg(0)
    pltpu.async_copy(sfts_hbm.at[CV[0]], G[0], gsem)

    def _body(k, P, Q, R):
        pltpu.make_async_copy(sfts_hbm.at[CV[P]], G[P], gsem).wait()
        wait_fg(P)

        @pl.when(k + 1 < NCHUNK)
        def _():
            wait_idx(k + 1, Q)
            issue_fg(Q)

        @pl.when(k + 2 < NCHUNK)
        def _():
            issue_idx(k + 2, R)

        @pl.when(k >= 1)
        def _():
            wait_scatter(R)

        @pl.when(k + 1 < NCHUNK)
        def _():
            pltpu.async_copy(sfts_hbm.at[CV[Q]], G[Q], gsem)

        for g in range(C // L):
            sl = pl.ds(g * L, L)
            RS[P][sl] = RV[P][sl]

        def _grp(g, _):
            sl16 = pl.ds(g * L, L)
            e = FA[P][sl16] + FB[P][sl16]
            e = jnp.where(e >= 0.0, e, 0.2 * e) - c_shift
            exvec = jnp.exp(e)
            plsc.store_scatter(X16[P], [iota16 + g * L, zeros16i], exvec)
            for lane in range(L):
                wv = jnp.full((L,), exvec[lane], jnp.float32)
                ei = g * L + lane
                for j in range(D // L):
                    sl = pl.ds(j * L, L)
                    G[P][ei, sl] = G[P][ei, sl] * wv
            return 0
        lax.fori_loop(0, C // L, _grp, 0)

        pltpu.async_copy(G[P], vals_sh.at[RS[P]], SS[P], add=True)
        pltpu.async_copy(X16[P], den16_sh.at[RS[P]], SS[P], add=True)

    def _full(k, _):
        for ph in range(3):
            @pl.when(k % 3 == ph)
            def _():
                _body(k, ph, (ph + 1) % 3, (ph + 2) % 3)
        return 0
    lax.fori_loop(0, NCHUNK, _full, 0)

    wait_scatter((NCHUNK - 1) % 3)
    plsc.subcore_barrier()

    drows = NP // NS
    for kk in range(drows // C):
        pltpu.sync_copy(den16_sh.at[pl.ds(sid * drows + kk * C, C), :], x0)
        for g in range(C // L):
            dv = plsc.load_gather(x0, [iota16 + g * L, zeros16i])
            for lane in range(L):
                ei = g * L + lane
                wv = jnp.full((L,), dv[lane], jnp.float32)
                for j in range(D // L):
                    g0[ei, pl.ds(j * L, L)] = wv

        @pl.when(core == 0)
        def _():
            pltpu.sync_copy(
                g0, den0_out.at[pl.ds(sid * drows + kk * C, C), :])

        @pl.when(core == 1)
        def _():
            pltpu.sync_copy(
                g0, den1_out.at[pl.ds(sid * drows + kk * C, C), :])

    @pl.when(core == 0)
    def _():
        pltpu.sync_copy(vals_sh.at[pl.ds(sid * drows, drows), :],
                        vals0_out.at[pl.ds(sid * drows, drows), :])

    @pl.when(core == 1)
    def _():
        pltpu.sync_copy(vals_sh.at[pl.ds(sid * drows, drows), :],
                        vals1_out.at[pl.ds(sid * drows, drows), :])


_sc1b = functools.partial(
    pl.kernel, _sc1b_body,
    compiler_params=_SC_PARAMS,
    out_type=(
        jax.ShapeDtypeStruct((NP, H), jnp.float32),
        jax.ShapeDtypeStruct((NP, H), jnp.float32),
        jax.ShapeDtypeStruct((NP, H), jnp.float32),
        jax.ShapeDtypeStruct((NP, H), jnp.float32),
    ),
    mesh=_MESH,
    scratch_types=(
        pltpu.VMEM((C,), jnp.int32),
        pltpu.VMEM((C,), jnp.int32),
        pltpu.VMEM((C,), jnp.int32),
        pltpu.VMEM((C,), jnp.int32),
        pltpu.VMEM((C,), jnp.int32),
        pltpu.VMEM((C,), jnp.int32),
        pltpu.VMEM((C,), jnp.float32),
        pltpu.VMEM((C,), jnp.float32),
        pltpu.VMEM((C,), jnp.float32),
        pltpu.VMEM((C,), jnp.float32),
        pltpu.VMEM((C,), jnp.float32),
        pltpu.VMEM((C,), jnp.float32),
        pltpu.VMEM((L,), jnp.float32),
        pltpu.VMEM((C,), jnp.int32),
        pltpu.VMEM((C,), jnp.int32),
        pltpu.VMEM((C,), jnp.int32),
        pltpu.VMEM((C, D), jnp.float32),
        pltpu.VMEM((C, D), jnp.float32),
        pltpu.VMEM((C, D), jnp.float32),
        pltpu.VMEM((C, L), jnp.float32),
        pltpu.VMEM((C, L), jnp.float32),
        pltpu.VMEM((C, L), jnp.float32),
        pltpu.VMEM((NP // NS,), jnp.float32),
        pltpu.VMEM_SHARED((NP, H), jnp.float32),
        pltpu.VMEM_SHARED((NP, L), jnp.float32),
        pltpu.SemaphoreType.DMA,
        pltpu.SemaphoreType.DMA,
        pltpu.SemaphoreType.DMA,
        pltpu.SemaphoreType.DMA,
        pltpu.SemaphoreType.DMA,
        pltpu.SemaphoreType.DMA,
    ),
)()


def _tc2_body(v0_ref, v1_ref, de0_ref, de1_ref, bias_ref, out_ref):
    v = v0_ref[...] + v1_ref[...]
    d = de0_ref[...] + de1_ref[...]
    o = v * jnp.where(d != 0.0, 1.0 / d, 0.0) + bias_ref[...]
    out_ref[0] = jnp.where(o > 0.0, o, jnp.exp(jnp.minimum(o, 0.0)) - 1.0)


def _tc2(vals0, vals1, den0, den1, bias2d):
    return pl.pallas_call(
        _tc2_body,
        grid=(N // TCB,),
        in_specs=[
            pl.BlockSpec((TCB, H), lambda i: (i, 0)),
            pl.BlockSpec((TCB, H), lambda i: (i, 0)),
            pl.BlockSpec((TCB, H), lambda i: (i, 0)),
            pl.BlockSpec((TCB, H), lambda i: (i, 0)),
            pl.BlockSpec((1, H), lambda i: (0, 0)),
        ],
        out_specs=pl.BlockSpec((1, TCB, H), lambda i: (0, i, 0)),
        out_shape=jax.ShapeDtypeStruct((1, N, H), jnp.float32),
    )(vals0, vals1, den0, den1, bias2d)


def kernel(seq, edge_index, training, W, a1, b1, a2, b2, bias_zero):
    x = seq[0]
    rows = edge_index[0]
    cols = edge_index[1]
    A = jnp.concatenate([a1, a2], axis=1)
    b2d = jnp.concatenate([b1, b2]).reshape(1, 2)
    sfts, f12, m12 = _tc1(x, W, A, b2d)
    f1 = f12[:, 0]
    f2 = f12[:, 1]
    mpad = jnp.pad(m12.reshape(2), (0, L - 2))
    vals0, vals1, den0, den1 = _sc1b(rows, cols, f1, f2, mpad, sfts)
    return _tc2(vals0[:N], vals1[:N], den0[:N], den1[:N],
                bias_zero.reshape(1, H))
